# baseline (speedup 1.0000x reference)
import functools

import jax
import jax.numpy as jnp
from jax import lax
from jax.experimental import pallas as pl
from jax.experimental.pallas import tpu as pltpu
from jax.experimental.pallas import tpu_sc as plsc

_NC = 2
_NS = 16
_NW = _NC * _NS
_LANES = 16

_SEG = 100
_SEGS_PER_CHUNK = 4


def _embed_body(seq, embed, n_flat, idx_hbm, table_hbm, pos_hbm, out_hbm,
                idx_v, rows_v, obuf_v, pos_v, gs0, gs1, os0, os1):
    chunk = _SEG * _SEGS_PER_CHUNK
    rows_per_chunk = chunk // seq
    per_worker = n_flat // _NW
    n_chunks = per_worker // chunk
    gsem = (gs0, gs1)
    osem = (os0, os1)

    wid = lax.axis_index("s") * _NC + lax.axis_index("c")
    seg_base = wid * (per_worker // _SEG)
    tok_base = wid * per_worker

    pltpu.sync_copy(pos_hbm.at[pl.ds(0, seq)], pos_v)

    def fire(k, b):
        seg0 = seg_base + k * _SEGS_PER_CHUNK
        pltpu.sync_copy(idx_hbm.at[pl.ds(seg0, _SEGS_PER_CHUNK)], idx_v.at[b])
        for j in range(_SEGS_PER_CHUNK):
            pltpu.async_copy(
                table_hbm.at[idx_v.at[b, j]],
                rows_v.at[b, pl.ds(j * _SEG, _SEG)],
                gsem[b],
            )

    def process(k, b, wait_out):
        pltpu.make_async_copy(
            table_hbm.at[pl.ds(0, chunk)], rows_v.at[b], gsem[b]).wait()
        if wait_out:
            pltpu.make_async_copy(
                obuf_v.at[b], out_hbm.at[pl.ds(0, chunk)], osem[b]).wait()

        def add_body(r, c2):
            for c in range(embed // _LANES):
                sl = pl.ds(c * _LANES, _LANES)
                p = pos_v[r, sl]
                for rep in range(rows_per_chunk):
                    obuf_v[b, rep * seq + r, sl] = rows_v[b, rep * seq + r, sl] + p
            return c2

        lax.fori_loop(0, seq, add_body, 0, unroll=2)
        pltpu.async_copy(
            obuf_v.at[b],
            out_hbm.at[pl.ds(tok_base + k * chunk, chunk)],
            osem[b],
        )

    fire(0, 0)
    fire(1, 1)
    process(0, 0, False)
    fire(2, 0)
    process(1, 1, False)
    fire(3, 1)

    def loop_body(j, carry):
        for b in range(2):
            k = 2 * j + 2 + b
            process(k, b, True)
            fire(k + 2, b)
        return carry

    lax.fori_loop(0, (n_chunks - 4) // 2, loop_body, 0)

    process(n_chunks - 2, 0, True)
    process(n_chunks - 1, 1, True)
    pltpu.make_async_copy(obuf_v.at[0], out_hbm.at[pl.ds(0, chunk)], os0).wait()
    pltpu.make_async_copy(obuf_v.at[1], out_hbm.at[pl.ds(0, chunk)], os1).wait()


def kernel(token_ids, text_table, pos_table):
    batch, seq = token_ids.shape
    vocab, embed = text_table.shape
    n_flat = batch * seq
    chunk = _SEG * _SEGS_PER_CHUNK

    idx_flat = jnp.reshape(token_ids.astype(jnp.int32), (n_flat // _SEG, _SEG))

    mesh = plsc.VectorSubcoreMesh(core_axis_name="c", subcore_axis_name="s")
    body = functools.partial(_embed_body, seq, embed, n_flat)
    out = pl.kernel(
        body,
        out_type=jax.ShapeDtypeStruct((n_flat, embed), jnp.float32),
        mesh=mesh,
        scratch_types=[
            pltpu.VMEM((2, _SEGS_PER_CHUNK, _SEG), jnp.int32),
            pltpu.VMEM((2, chunk, embed), jnp.float32),
            pltpu.VMEM((2, chunk, embed), jnp.float32),
            pltpu.VMEM((seq, embed), jnp.float32),
            pltpu.SemaphoreType.DMA,
            pltpu.SemaphoreType.DMA,
            pltpu.SemaphoreType.DMA,
            pltpu.SemaphoreType.DMA,
        ],
        compiler_params=pltpu.CompilerParams(use_tc_tiling_on_sc=False),
        name="sc_embed_lookup",
    )(idx_flat, text_table, pos_table)
    return jnp.reshape(out, (batch, seq, embed))
